# Initial kernel scaffold; baseline (speedup 1.0000x reference)
import functools

import jax
import jax.numpy as jnp
from jax import lax
from jax.experimental import pallas as pl
from jax.experimental.pallas import tpu as pltpu
from jax.experimental.pallas import tpu_sc as plsc

N_ENSEMBLE = 10
VOCAB = 100000
DIM = 64
B, L = 4096, 50
N = B * L

_info = plsc.get_sparse_core_info()
NC, NS = _info.num_cores, _info.num_subcores
NW = NC * NS
PER_W = N // NW
CH = 128
NCH = PER_W // CH

_mesh = plsc.VectorSubcoreMesh(core_axis_name="c", subcore_axis_name="s")


@functools.partial(
    pl.kernel,
    mesh=_mesh,
    out_type=tuple(
        jax.ShapeDtypeStruct((N, DIM), jnp.float32) for _ in range(N_ENSEMBLE)
    ),
    scratch_types=[
        pltpu.VMEM((NCH, CH), jnp.int32),
        pltpu.VMEM((CH, DIM), jnp.float32),
        pltpu.SemaphoreType.DMA,
    ],
)
def _ensemble_lookup(idx_hbm, tab_hbm, *rest):
    outs = rest[:N_ENSEMBLE]
    idx_v, rows_v, sem = rest[N_ENSEMBLE:]
    wid = lax.axis_index("s") * NC + lax.axis_index("c")
    base = wid * PER_W
    pltpu.sync_copy(idx_hbm.at[wid], idx_v)

    for t in range(N_ENSEMBLE):
        def chunk_body(c, _, t=t):
            pltpu.async_copy(tab_hbm.at[t].at[idx_v.at[c]], rows_v, sem).wait()

            def row_body(r, _):
                for j in range(DIM // 16):
                    s = pl.ds(j * 16, 16)
                    rows_v[r, s] = rows_v[r, s] * 8.0
                return 0

            lax.fori_loop(0, CH, row_body, 0, unroll=2)
            pltpu.sync_copy(rows_v, outs[t].at[pl.ds(base + c * CH, CH)])
            return 0

        lax.fori_loop(0, NCH, chunk_body, 0)


def kernel(indices, tables):
    idx = indices.astype(jnp.int32).reshape(NW, NCH, CH)
    outs = _ensemble_lookup(idx, tables)
    return tuple(o.reshape(B, L, DIM) for o in outs)


# speedup vs baseline: 4.7245x; 4.7245x over previous
import functools

import jax
import jax.numpy as jnp
from jax import lax
from jax.experimental import pallas as pl
from jax.experimental.pallas import tpu as pltpu
from jax.experimental.pallas import tpu_sc as plsc

N_ENSEMBLE = 10
VOCAB = 100000
DIM = 64
B, L = 4096, 50
N = B * L

_info = plsc.get_sparse_core_info()
NC, NS = _info.num_cores, _info.num_subcores
NW = NC * NS
PER_W = N // NW
CH = 128
NCH = PER_W // CH

_mesh = plsc.VectorSubcoreMesh(core_axis_name="c", subcore_axis_name="s")


@functools.partial(
    pl.kernel,
    mesh=_mesh,
    compiler_params=pltpu.CompilerParams(use_tc_tiling_on_sc=False),
    out_type=tuple(
        jax.ShapeDtypeStruct((N, DIM), jnp.float32) for _ in range(N_ENSEMBLE)
    ),
    scratch_types=[
        pltpu.VMEM((NCH, CH), jnp.int32),
        pltpu.VMEM((CH, DIM), jnp.float32),
        pltpu.SemaphoreType.DMA,
    ],
)
def _ensemble_lookup(idx_hbm, tab_hbm, *rest):
    outs = rest[:N_ENSEMBLE]
    idx_v, rows_v, sem = rest[N_ENSEMBLE:]
    wid = lax.axis_index("s") * NC + lax.axis_index("c")
    base = wid * PER_W
    pltpu.sync_copy(idx_hbm.at[wid], idx_v)

    for t in range(N_ENSEMBLE):
        def chunk_body(c, _, t=t):
            pltpu.async_copy(tab_hbm.at[t].at[idx_v.at[c]], rows_v, sem).wait()

            def row_body(r, _):
                for j in range(DIM // 16):
                    s = pl.ds(j * 16, 16)
                    rows_v[r, s] = rows_v[r, s] * 8.0
                return 0

            lax.fori_loop(0, CH, row_body, 0, unroll=2)
            pltpu.sync_copy(rows_v, outs[t].at[pl.ds(base + c * CH, CH)])
            return 0

        lax.fori_loop(0, NCH, chunk_body, 0)


def kernel(indices, tables):
    idx = indices.astype(jnp.int32).reshape(NW, NCH, CH)
    outs = _ensemble_lookup(idx, tables)
    return tuple(o.reshape(B, L, DIM) for o in outs)
